# Initial kernel scaffold; baseline (speedup 1.0000x reference)
import jax
import jax.numpy as jnp
from jax.experimental import pallas as pl


def kernel(s, v, dir_ij, Wij, senders, receivers, W_int1, b_int1, W_int2, b_int2, W_vmix, W_mix1, b_mix1, W_mix2, b_mix2):
    raise NotImplementedError("write your pallas kernel here")


# speedup vs baseline: 3.3345x; 3.3345x over previous
import functools

import jax
import jax.numpy as jnp
from jax import lax
from jax.experimental import pallas as pl
from jax.experimental.pallas import tpu as pltpu
from jax.experimental.pallas import tpu_sc as plsc

H = 128
H3 = 3 * H
EPS = 1e-08

NW = 32
NB = 160
NBKT = 63
ROUNDS = 2
NOUT = NBKT * NB
CHW = 2000


def _clip(x):
    return jnp.clip(x, -100.0, 100.0)


def _bucket(snd):
    return ((snd >> 5) * 13108) >> 16


def _mlp1(s2d, W1, b1, W2, b2):
    n = s2d.shape[0]
    bs = 400

    def body(s_ref, w1_ref, b1_ref, w2_ref, b2_ref, o_ref):
        h = jnp.dot(s_ref[...], w1_ref[...], preferred_element_type=jnp.float32)
        h = h + b1_ref[...]
        h = h * jax.nn.sigmoid(h)
        o = jnp.dot(h, w2_ref[...], preferred_element_type=jnp.float32)
        o_ref[...] = o + b2_ref[...]

    return pl.pallas_call(
        body,
        grid=(n // bs,),
        in_specs=[
            pl.BlockSpec((bs, H), lambda i: (i, 0)),
            pl.BlockSpec((H, H), lambda i: (0, 0)),
            pl.BlockSpec((1, H), lambda i: (0, 0)),
            pl.BlockSpec((H, H3), lambda i: (0, 0)),
            pl.BlockSpec((1, H3), lambda i: (0, 0)),
        ],
        out_specs=pl.BlockSpec((bs, H3), lambda i: (i, 0)),
        out_shape=jax.ShapeDtypeStruct((n, H3), jnp.float32),
    )(s2d, W1, b1.reshape(1, H), W2, b2.reshape(1, H3))


def _sc_mesh():
    return plsc.VectorSubcoreMesh(core_axis_name="c", subcore_axis_name="s")


def _wid():
    return lax.axis_index("s") * 2 + lax.axis_index("c")


def _sc_hist(senders):
    e = senders.shape[0]
    epw = e // NW

    @functools.partial(
        pl.kernel,
        out_type=jax.ShapeDtypeStruct((NW * 64,), jnp.int32),
        mesh=_sc_mesh(),
        scratch_types=[
            pltpu.VMEM((64,), jnp.int32),
            pltpu.VMEM((CHW,), jnp.int32),
        ],
    )
    def k(snd_hbm, out_hbm, hist, pbuf):
        wid = _wid()
        iot = lax.iota(jnp.int32, 16)
        one_i = jnp.ones((16,), jnp.int32)
        zero_i = jnp.zeros((16,), jnp.int32)
        for g in range(4):
            hist[pl.ds(g * 16, 16)] = zero_i
        for ch in range(epw // CHW):
            pltpu.sync_copy(
                snd_hbm.at[pl.ds(wid * epw + ch * CHW, CHW)], pbuf)

            def vec_body(i, c):
                sv = pbuf[pl.ds(i * 16, 16)]
                for j in range(16):
                    bb = _bucket(sv[j])
                    oh = jnp.where(iot == (bb & 15), one_i, zero_i)
                    plsc.addupdate(hist.at[pl.ds((bb >> 4) * 16, 16)], oh)
                return c

            lax.fori_loop(0, CHW // 16, vec_body, 0)
        pltpu.sync_copy(hist, out_hbm.at[pl.ds(wid * 64, 64)])

    return k(senders)


def _sc_scatter(senders, meta2d, counts):
    e = senders.shape[0]
    epw = e // NW

    @functools.partial(
        pl.kernel,
        out_type=jax.ShapeDtypeStruct((e + NBKT * 8 + 16, 16), jnp.float32),
        mesh=_sc_mesh(),
        scratch_types=[
            pltpu.VMEM((NW * 64,), jnp.int32),
            pltpu.VMEM((CHW,), jnp.int32),
            pltpu.SMEM((64,), jnp.int32),
            pltpu.SemaphoreType.DMA,
        ],
    )
    def k(snd_hbm, meta_hbm, cnt_hbm, out_hbm, cbuf, pbuf, smem, sem):
        wid = _wid()
        zero_i = jnp.zeros((16,), jnp.int32)
        pltpu.sync_copy(cnt_hbm, cbuf)
        T = [zero_i] * 4
        PS = [zero_i] * 4
        for t in range(NW):
            before = t < wid
            for g in range(4):
                r = cbuf[pl.ds(t * 64 + g * 16, 16)]
                T[g] = T[g] + r
                PS[g] = PS[g] + jnp.where(before, r, zero_i)
        s_run = jnp.int32(0)
        for b in range(NBKT):
            g, l = b >> 4, b & 15
            smem[b] = s_run + PS[g][l]
            s_run = s_run + (((T[g][l] + 7) >> 3) << 3)

        def drain():
            pltpu.make_async_copy(
                meta_hbm.at[0], out_hbm.at[0], sem).wait()

        for ch in range(epw // CHW):
            base_c = wid * epw + ch * CHW
            pltpu.sync_copy(snd_hbm.at[pl.ds(base_c, CHW)], pbuf)

            def vec_body(i, c):
                first = (ch == 0) & (i == 0)

                @pl.when(jnp.logical_not(first))
                def _():
                    for _ in range(16):
                        drain()

                sv = pbuf[pl.ds(i * 16, 16)]
                for j in range(16):
                    bb = _bucket(sv[j])
                    cur = smem[bb]
                    smem[bb] = cur + 1
                    src = base_c + i * 16 + j
                    pltpu.async_copy(
                        meta_hbm.at[src], out_hbm.at[cur], sem)
                return c

            lax.fori_loop(0, CHW // 16, vec_body, 0)
        for _ in range(16):
            drain()

    return k(senders, meta2d, counts)


def _sc_main(sorted2d, counts, x1d, v1d, w1d):
    @functools.partial(
        pl.kernel,
        out_type=jax.ShapeDtypeStruct((NOUT, 4 * H), jnp.float32),
        mesh=_sc_mesh(),
        scratch_types=[
            pltpu.VMEM((NB, 4 * H), jnp.float32),
            pltpu.VMEM((NW * 64,), jnp.int32),
            pltpu.VMEM((16, 16), jnp.float32),
            pltpu.VMEM((H3,), jnp.float32),
            pltpu.VMEM((H3,), jnp.float32),
            pltpu.VMEM((H3,), jnp.float32),
            pltpu.VMEM((H3,), jnp.float32),
            pltpu.VMEM((H3,), jnp.float32),
            pltpu.VMEM((H3,), jnp.float32),
            pltpu.SMEM((128,), jnp.int32),
            pltpu.SemaphoreType.DMA,
            pltpu.SemaphoreType.DMA,
        ],
    )
    def k(rec_hbm, cnt_hbm, x_hbm, v_hbm, w_hbm, out_hbm,
          acc, cbuf, recbuf, w0, w1, x0, x1, v0, v1, smem, sem0, sem1):
        wid = _wid()
        zero16 = jnp.zeros((16,), jnp.float32)
        zero_i = jnp.zeros((16,), jnp.int32)
        wslot = (w0, w1)
        xslot = (x0, x1)
        vslot = (v0, v1)
        sems = (sem0, sem1)

        pltpu.sync_copy(cnt_hbm, cbuf)
        T = [zero_i] * 4
        for t in range(NW):
            for g in range(4):
                T[g] = T[g] + cbuf[pl.ds(t * 64 + g * 16, 16)]
        s_run = jnp.int32(0)
        for b in range(NBKT):
            smem[b] = s_run >> 3
            smem[64 + b] = T[b >> 4][b & 15]
            s_run = s_run + (((T[b >> 4][b & 15] + 7) >> 3) << 3)

        def fire(rec, s):
            eid = jnp.int32(rec[5])
            rcv = jnp.int32(rec[1])
            pltpu.async_copy(w_hbm.at[pl.ds(eid * H3, H3)], wslot[s], sems[s])
            pltpu.async_copy(x_hbm.at[pl.ds(rcv * H3, H3)], xslot[s], sems[s])
            pltpu.async_copy(v_hbm.at[pl.ds(rcv * H3, H3)], vslot[s], sems[s])

        def wait_slot(s):
            for buf in (wslot[s], xslot[s], vslot[s]):
                pltpu.make_async_copy(
                    w_hbm.at[pl.ds(0, H3)], buf, sems[s]).wait()

        def compute(rec, s, base):
            row = jnp.int32(rec[0]) - base
            dvecs = [jnp.full((16,), rec[2 + kk], jnp.float32)
                     for kk in range(3)]
            ws, xs, vs = wslot[s], xslot[s], vslot[s]
            for c in range(8):
                co = c * 16
                wv0 = ws[pl.ds(co, 16)]
                wv1 = ws[pl.ds(H + co, 16)]
                wv2 = ws[pl.ds(2 * H + co, 16)]
                xv0 = xs[pl.ds(co, 16)]
                xv1 = xs[pl.ds(H + co, 16)]
                xv2 = xs[pl.ds(2 * H + co, 16)]
                dv1 = wv1 * xv1
                dv2 = wv2 * xv2
                plsc.addupdate(acc.at[row, pl.ds(co, 16)], wv0 * xv0)
                for kk in range(3):
                    vj = vs[pl.ds(kk * H + co, 16)]
                    plsc.addupdate(
                        acc.at[row, pl.ds(H + kk * H + co, 16)],
                        dv1 * dvecs[kk] + dv2 * vj)

        for r in range(ROUNDS):
            b = r * NW + wid

            @pl.when(b < NBKT)
            def _():
                base = b * NB

                def zrow(i, c):
                    for cc in range(4 * H // 16):
                        acc[i, pl.ds(cc * 16, 16)] = zero16
                    return c

                lax.fori_loop(0, NB, zrow, 0)
                lo8 = smem[b]
                cnt = smem[64 + b]
                nch = (cnt + 15) >> 4

                def chunk_body(ch, c):
                    cbase = (lo8 + ch * 2) * 8
                    pltpu.sync_copy(
                        rec_hbm.at[pl.ds(cbase, 16)], recbuf)
                    rec0 = recbuf[0, pl.ds(0, 16)]
                    fire(rec0, 0)
                    for j in range(16):
                        recj = recbuf[j, pl.ds(0, 16)]
                        if j < 15:
                            @pl.when(ch * 16 + j + 1 < cnt)
                            def _():
                                fire(recbuf[j + 1, pl.ds(0, 16)],
                                     (j + 1) & 1)

                        @pl.when(ch * 16 + j < cnt)
                        def _():
                            wait_slot(j & 1)
                            compute(recj, j & 1, base)

                    return c

                lax.fori_loop(0, nch, chunk_body, 0)
                pltpu.sync_copy(acc, out_hbm.at[pl.ds(base, NB)])

    return k(sorted2d, counts, x1d, v1d, w1d)


def _update(s2d, v2d, dsum, dvsum, W_vmix, W_mix1, b_mix1, W_mix2, b_mix2):
    n = s2d.shape[0]
    bs = 400

    def body(s_ref, v_ref, ds_ref, dv_ref, wv_ref, w1_ref, b1_ref, w2_ref,
             b2_ref, so_ref, vo_ref):
        s1 = s_ref[...] + _clip(ds_ref[...])
        v1 = v_ref[...] + _clip(dv_ref[...])
        wv = wv_ref[...]
        v1k = [v1[:, kk * H:(kk + 1) * H] for kk in range(3)]
        vm = [jnp.dot(vk, wv, preferred_element_type=jnp.float32) for vk in v1k]
        v_l = [m[:, :H] for m in vm]
        v_r = [m[:, H:] for m in vm]
        nsq = v_r[0] * v_r[0] + v_r[1] * v_r[1] + v_r[2] * v_r[2]
        v_norm = jnp.sqrt(nsq + EPS)
        w1 = w1_ref[...]
        h = (jnp.dot(s1, w1[:H, :], preferred_element_type=jnp.float32)
             + jnp.dot(v_norm, w1[H:, :], preferred_element_type=jnp.float32)
             + b1_ref[...])
        h = h * jax.nn.sigmoid(h)
        m = jnp.dot(h, w2_ref[...], preferred_element_type=jnp.float32)
        m = m + b2_ref[...]
        ds2 = m[:, :H]
        dvu_g = m[:, H:2 * H]
        dsv_g = m[:, 2 * H:]
        dot_rl = v_r[0] * v_l[0] + v_r[1] * v_l[1] + v_r[2] * v_l[2]
        so_ref[...] = s1 + _clip(ds2 + dsv_g * dot_rl)
        vo_ref[...] = jnp.concatenate(
            [v1k[kk] + _clip(v_l[kk] * dvu_g) for kk in range(3)], axis=1)

    return pl.pallas_call(
        body,
        grid=(n // bs,),
        in_specs=[
            pl.BlockSpec((bs, H), lambda i: (i, 0)),
            pl.BlockSpec((bs, H3), lambda i: (i, 0)),
            pl.BlockSpec((bs, H), lambda i: (i, 0)),
            pl.BlockSpec((bs, H3), lambda i: (i, 0)),
            pl.BlockSpec((H, 2 * H), lambda i: (0, 0)),
            pl.BlockSpec((2 * H, H), lambda i: (0, 0)),
            pl.BlockSpec((1, H), lambda i: (0, 0)),
            pl.BlockSpec((H, H3), lambda i: (0, 0)),
            pl.BlockSpec((1, H3), lambda i: (0, 0)),
        ],
        out_specs=[
            pl.BlockSpec((bs, H), lambda i: (i, 0)),
            pl.BlockSpec((bs, H3), lambda i: (i, 0)),
        ],
        out_shape=[
            jax.ShapeDtypeStruct((n, H), jnp.float32),
            jax.ShapeDtypeStruct((n, H3), jnp.float32),
        ],
    )(s2d, v2d, dsum, dvsum, W_vmix, W_mix1, b_mix1.reshape(1, H), W_mix2,
      b_mix2.reshape(1, H3))


def kernel(s, v, dir_ij, Wij, senders, receivers, W_int1, b_int1, W_int2,
           b_int2, W_vmix, W_mix1, b_mix1, W_mix2, b_mix2):
    n = s.shape[0]
    e = senders.shape[0]
    s2d = s.reshape(n, H)
    v2d = v.reshape(n, H3)
    w1d = Wij.reshape(e * H3)

    x2d = _mlp1(s2d, W_int1, b_int1, W_int2, b_int2)

    meta = jnp.concatenate(
        [senders.astype(jnp.float32)[:, None],
         receivers.astype(jnp.float32)[:, None], dir_ij,
         jnp.arange(e, dtype=jnp.float32)[:, None],
         jnp.zeros((e, 10), jnp.float32)], axis=1)
    counts = _sc_hist(senders)
    sorted2d = _sc_scatter(senders, meta, counts)
    agg = _sc_main(sorted2d, counts, x2d.reshape(n * H3), v2d.reshape(n * H3),
                   w1d)
    dsum = agg[:n, :H]
    dvsum = agg[:n, H:]

    s_out, v_out = _update(s2d, v2d, dsum, dvsum, W_vmix, W_mix1, b_mix1,
                           W_mix2, b_mix2)
    return (s_out.reshape(n, 1, H), v_out.reshape(n, 3, H))
